# baseline (speedup 1.0000x reference)
import jax
import jax.numpy as jnp
import numpy as np
from jax.experimental import pallas as pl
from jax.experimental.pallas import tpu as pltpu

_TINY = np.float32(np.finfo(np.float32).tiny)
_ROT_A = (13, 15, 26, 6)
_ROT_B = (17, 29, 16, 24)


def _threefry_bits(n):
    k0 = jnp.uint32(0)
    k1 = jnp.uint32(42)
    k2 = jnp.uint32(0 ^ 42 ^ 0x1BD11BDA)

    def rotl(x, r):
        return (x << jnp.uint32(r)) | (x >> jnp.uint32(32 - r))

    x0 = jnp.zeros_like(n)
    x1 = n + k1
    sched = (
        (_ROT_A, k1, k2, 1),
        (_ROT_B, k2, k0, 2),
        (_ROT_A, k0, k1, 3),
        (_ROT_B, k1, k2, 4),
        (_ROT_A, k2, k0, 5),
    )
    for rots, ka, kb, i in sched:
        for r in rots:
            x0 = x0 + x1
            x1 = rotl(x1, r)
            x1 = x1 ^ x0
        x0 = x0 + ka
        x1 = x1 + kb + jnp.uint32(i)
    return x0 ^ x1


def _row_kernel(p_ref, out_ref, *, v, sub, cols):
    r = pl.program_id(0)
    p = p_ref[0]

    flat = (jax.lax.broadcasted_iota(jnp.int32, (sub, cols), 0) * cols
            + jax.lax.broadcasted_iota(jnp.int32, (sub, cols), 1))
    n = (flat + r * v).astype(jnp.uint32)

    bits = _threefry_bits(n)
    fbits = (bits >> jnp.uint32(9)) | jnp.uint32(0x3F800000)
    floats = jax.lax.bitcast_convert_type(fbits, jnp.float32) - jnp.float32(1.0)
    u = jnp.maximum(jnp.float32(_TINY), floats)
    gumbel = -jnp.log(-jnp.log(u))

    total = jnp.sum(p)
    logits = jnp.log(p / total + jnp.float32(1e-30))
    score = gumbel + logits

    m = jnp.max(score)
    idx = jnp.min(jnp.where(score == m, flat, jnp.int32(2**31 - 1)))
    out_ref[...] = jnp.reshape(idx, (1, 1, 1))


def kernel(probabilities):
    b, s, v = probabilities.shape
    rows = b * s
    sub = 8
    assert v % sub == 0
    cols = v // sub
    p3 = probabilities.reshape(rows, sub, cols)

    out = pl.pallas_call(
        lambda p_ref, out_ref: _row_kernel(p_ref, out_ref, v=v, sub=sub,
                                           cols=cols),
        grid=(rows,),
        in_specs=[pl.BlockSpec((1, sub, cols), lambda i: (i, 0, 0))],
        out_specs=pl.BlockSpec((1, 1, 1), lambda i: (i, 0, 0)),
        out_shape=jax.ShapeDtypeStruct((rows, 1, 1), jnp.int32),
        compiler_params=pltpu.CompilerParams(
            dimension_semantics=("parallel",)),
    )(p3)
    return out.reshape(b, s)
